# Initial kernel scaffold; baseline (speedup 1.0000x reference)
import jax
import jax.numpy as jnp
from jax.experimental import pallas as pl


def kernel(table, input):
    raise NotImplementedError("write your pallas kernel here")


# speedup vs baseline: 1.7678x; 1.7678x over previous
import functools

import jax
import jax.numpy as jnp
from jax import lax
from jax.experimental import pallas as pl
from jax.experimental.pallas import tpu as pltpu
from jax.experimental.pallas import tpu_sc as plsc

NC = 2
NS = 16
NW = NC * NS

NB = 8


def _sc_gather(idx, table):
    b, h = idx.shape
    d = table.shape[1]
    samples_per_w = b // NW
    chunks = samples_per_w // NB

    mesh = plsc.VectorSubcoreMesh(
        core_axis_name="c", subcore_axis_name="s", num_cores=NC, num_subcores=NS
    )

    @functools.partial(
        pl.kernel,
        out_type=jax.ShapeDtypeStruct((b, h, d), jnp.float32),
        mesh=mesh,
        scratch_types=[
            pltpu.VMEM((NB, h), jnp.int32),
            pltpu.VMEM((NB, h, d), jnp.float32),
            pltpu.SemaphoreType.DMA,
        ],
        compiler_params=pltpu.CompilerParams(use_tc_tiling_on_sc=False),
    )
    def run(idx_hbm, table_hbm, out_hbm, idx_v, rows_v, sem):
        wid = lax.axis_index("s") * NC + lax.axis_index("c")
        base = wid * samples_per_w

        @pl.loop(0, chunks)
        def _(i):
            b0 = base + i * NB
            pltpu.sync_copy(idx_hbm.at[pl.ds(b0, NB)], idx_v)
            copies = []
            for j in range(NB):
                copies.append(
                    pltpu.async_copy(table_hbm.at[idx_v.at[j]], rows_v.at[j], sem)
                )
            for c in copies:
                c.wait()
            pltpu.sync_copy(rows_v, out_hbm.at[pl.ds(b0, NB)])

    return run(idx, table)


def kernel(table, input):
    idx = input.astype(jnp.int32)
    return _sc_gather(idx, table)
